# baseline (speedup 1.0000x reference)
import functools

import jax
import jax.numpy as jnp
from jax import lax
from jax.experimental import pallas as pl
from jax.experimental.pallas import tpu as pltpu, tpu_sc as plsc

B = 16384
L = 200
LH = L // 2
BL = B * L

NC = 2
NS = 16
NW = NC * NS
LANES = 16

BPW = B // NW
CTX_PAD = 16
TV = 1000
LOC_VP = 1024000


def _proj_body(xt_ref, w_ref, b_ref, o_ref):
    y = lax.dot_general(
        xt_ref[...], w_ref[...], (((0,), (1,)), ((), ())),
        preferred_element_type=jnp.float32,
    )
    o_ref[...] = y + b_ref[...]


def _project(xt, w, b_row, blk):
    d_in, v = xt.shape
    return pl.pallas_call(
        _proj_body,
        grid=(v // blk,),
        in_specs=[
            pl.BlockSpec((d_in, blk), lambda i: (0, i)),
            pl.BlockSpec((32, d_in), lambda i: (0, 0)),
            pl.BlockSpec((1, 32), lambda i: (0, 0)),
        ],
        out_specs=pl.BlockSpec((blk, 32), lambda i: (i, 0)),
        out_shape=jax.ShapeDtypeStruct((v, 32), jnp.float32),
    )(xt, w, b_row)


def _retile_body(x_ref, o_ref):
    o_ref[...] = x_ref[...].T


def _retile(z):
    x2 = z.reshape(z.shape[0] * B * 128 // 128, 128)
    o2 = pl.pallas_call(
        _retile_body,
        grid=(L // 4, 4),
        in_specs=[pl.BlockSpec((4096, 128), lambda i, j: (i * 4 + j, 0))],
        out_specs=pl.BlockSpec((128, 4096), lambda i, j: (i, j)),
        out_shape=jax.ShapeDtypeStruct((L * 32, B), jnp.float32),
    )(x2)
    return o2.reshape(L, 32, B).transpose(2, 0, 1)


def _retile_body_al(x_ref, prev_ref, o_ref):
    del prev_ref
    o_ref[...] = x_ref[...].T


def _retile_halves(z1, z2):
    o2 = None
    for half, z in ((0, z1), (1, z2)):
        x2 = z.reshape(LH * B * 32 // 128, 128)
        grid_i = LH // 4
        out_map = functools.partial(
            lambda h, i, j: (h * (LH // 4) + i, j), half)
        if half == 0:
            o2 = pl.pallas_call(
                _retile_body,
                grid=(grid_i, 4),
                in_specs=[
                    pl.BlockSpec((4096, 128), lambda i, j: (i * 4 + j, 0))],
                out_specs=pl.BlockSpec((128, 4096), out_map),
                out_shape=jax.ShapeDtypeStruct((L * 32, B), jnp.float32),
            )(x2)
        else:
            o2 = pl.pallas_call(
                _retile_body_al,
                grid=(grid_i, 4),
                in_specs=[
                    pl.BlockSpec((4096, 128), lambda i, j: (i * 4 + j, 0)),
                    pl.BlockSpec(memory_space=pltpu.MemorySpace.HBM),
                ],
                out_specs=pl.BlockSpec((128, 4096), out_map),
                out_shape=jax.ShapeDtypeStruct((L * 32, B), jnp.float32),
                input_output_aliases={1: 0},
            )(x2, o2)
    return o2.reshape(L, 32, B).transpose(2, 0, 1)


_MESH = plsc.VectorSubcoreMesh(core_axis_name="c", subcore_axis_name="s")
_SC_PARAMS = pltpu.CompilerParams(
    use_tc_tiling_on_sc=False, needs_layout_passes=False)

_SLAB = jax.ShapeDtypeStruct((L // 4, B, 128), jnp.float32)
_SLABH = jax.ShapeDtypeStruct((LH // 4, B, 128), jnp.float32)


def _slab_dst(out_hbm, l, b0, l_off=0):
    ll = l - l_off
    return out_hbm.at[ll // 4, pl.ds(b0, BPW), pl.ds(lax.rem(ll, 4) * 32, 32)]


def _ring_streams(streams, b0, l_lo, l_hi, l_off=0):
    def fire(l, h):
        for tokT, tab, _out, idx_v, rows_v, gs, _ws in streams:
            pltpu.sync_copy(tokT.at[l, pl.ds(b0, BPW)], idx_v.at[h])
            pltpu.async_copy(tab.at[idx_v.at[h]], rows_v.at[h], gs.at[h])

    def drain_fire_out(l, h):
        for _tokT, tab, out, idx_v, rows_v, gs, ws in streams:
            pltpu.make_async_copy(
                tab.at[idx_v.at[h]], rows_v.at[h], gs.at[h]).wait()
            pltpu.async_copy(
                rows_v.at[h], _slab_dst(out, l, b0, l_off), ws.at[h])

    def wait_out(l, h):
        for _tokT, _tab, out, _idx_v, rows_v, _gs, ws in streams:
            pltpu.make_async_copy(
                rows_v.at[h], _slab_dst(out, l, b0, l_off), ws.at[h]).wait()

    for h in range(2):
        fire(l_lo + h, h)
    for h in range(2):
        drain_fire_out(l_lo + h, h)

    @pl.loop(l_lo + 2, l_hi, step=2)
    def _rows(i):
        for h in range(2):
            wait_out(i + h - 2, h)
            fire(i + h, h)
        for h in range(2):
            drain_fire_out(i + h, h)

    for h in range(2):
        wait_out(l_hi - 2 + h, h)


def _transpose_chunk16(rows, trows, c):
    giota = lax.iota(jnp.int32, 16)
    for d in range(CTX_PAD):
        dvec = jnp.full((16,), d, jnp.int32)
        for g in range(c // LANES):
            rvec = giota + (g * LANES)
            trows[d, pl.ds(g * LANES, LANES)] = plsc.load_gather(
                rows, [rvec, dvec])


@functools.partial(
    pl.kernel,
    mesh=_MESH,
    compiler_params=_SC_PARAMS,
    out_type=[
        _SLAB,
        _SLAB,
        jax.ShapeDtypeStruct((CTX_PAD, B), jnp.float32),
    ],
    scratch_types=[
        pltpu.VMEM_SHARED((TV, 32), jnp.float32),
        pltpu.VMEM_SHARED((TV, 32), jnp.float32),
        pltpu.VMEM_SHARED((304, CTX_PAD), jnp.float32),
        pltpu.VMEM((2, BPW), jnp.int32),
        pltpu.VMEM((2, BPW, 32), jnp.float32),
        pltpu.VMEM((2, BPW), jnp.int32),
        pltpu.VMEM((2, BPW, 32), jnp.float32),
        pltpu.VMEM((BPW,), jnp.int32),
        pltpu.VMEM((BPW,), jnp.int32),
        pltpu.VMEM((BPW,), jnp.int32),
        pltpu.VMEM((BPW, CTX_PAD), jnp.float32),
        pltpu.VMEM((CTX_PAD, BPW), jnp.float32),
        pltpu.SemaphoreType.DMA((2,)),
        pltpu.SemaphoreType.DMA((2,)),
        pltpu.SemaphoreType.DMA((2,)),
        pltpu.SemaphoreType.DMA((2,)),
        pltpu.SemaphoreType.DMA,
    ],
)
def _gather_time_act_ctx(ttT_hbm, atT_hbm, c0_hbm, c1_hbm,
                         p_time_hbm, p_act_hbm, p_ctx_hbm,
                         out_t_hbm, out_a_hbm, out_c_hbm,
                         ptime_sh, pact_sh, comb_sh,
                         t_idx, t_rows, a_idx, a_rows,
                         c0_v, c1_v, cidx_v, crows_v, ctr_v,
                         tg, to, ag, ao, csem):
    cid = lax.axis_index("c")
    sid = lax.axis_index("s")
    wid = sid * NC + cid
    b0 = wid * BPW

    @pl.when(sid == 0)
    def _stage():
        pltpu.sync_copy(p_time_hbm, ptime_sh)
        pltpu.sync_copy(p_act_hbm, pact_sh)
        pltpu.sync_copy(p_ctx_hbm, comb_sh)

    plsc.subcore_barrier()

    pltpu.sync_copy(c0_hbm.at[pl.ds(b0, BPW)], c0_v)
    pltpu.sync_copy(c1_hbm.at[pl.ds(b0, BPW)], c1_v)
    for j in range(BPW // LANES):
        sl = pl.ds(j * LANES, LANES)
        cidx_v[sl] = c0_v[sl] * 100 + c1_v[sl]
    pltpu.async_copy(comb_sh.at[cidx_v], crows_v, csem).wait()
    _transpose_chunk16(crows_v, ctr_v, BPW)
    pltpu.sync_copy(ctr_v, out_c_hbm.at[:, pl.ds(b0, BPW)])

    _ring_streams(
        [
            (ttT_hbm, ptime_sh, out_t_hbm, t_idx, t_rows, tg, to),
            (atT_hbm, pact_sh, out_a_hbm, a_idx, a_rows, ag, ao),
        ],
        b0, 0, L,
    )


def _make_loc_half(l0):
    @functools.partial(
        pl.kernel,
        mesh=_MESH,
        compiler_params=_SC_PARAMS,
        out_type=_SLABH,
        scratch_types=[
            pltpu.VMEM((2, BPW), jnp.int32),
            pltpu.VMEM((2, BPW, 32), jnp.float32),
            pltpu.SemaphoreType.DMA((2,)),
            pltpu.SemaphoreType.DMA((2,)),
        ],
    )
    def _gather_loc_half(ltT_hbm, p_loc_hbm, out_hbm, idx_v, rows_v, gs, os_):
        wid = lax.axis_index("s") * NC + lax.axis_index("c")
        b0 = wid * BPW
        _ring_streams(
            [(ltT_hbm, p_loc_hbm, out_hbm, idx_v, rows_v, gs, os_)],
            b0, l0, l0 + LH, l_off=l0,
        )

    return _gather_loc_half


_gather_loc_h0 = _make_loc_half(0)
_gather_loc_h1 = _make_loc_half(LH)


def kernel(context_tokens, time_tokens, loc_tokens, act_tokens,
           time_table, loc_table, act_table, age_table, gender_table,
           W_time, b_time, W_loc, b_loc, W_act, b_act):
    p_time = _project(time_table.T, W_time, b_time.reshape(1, 32), TV)
    p_act = _project(act_table.T, W_act, b_act.reshape(1, 32), TV)
    ltp = jnp.pad(loc_table.T, ((0, 0), (0, LOC_VP - 1000000)))
    p_loc = _project(ltp, W_loc, b_loc.reshape(1, 32), 12800)

    comb = jnp.zeros((304, CTX_PAD), jnp.float32)
    comb = comb.at[:300, :2].set(jnp.repeat(gender_table, 100, axis=0))
    comb = comb.at[:300, 2:6].set(jnp.tile(age_table, (3, 1)))

    ttT = time_tokens.T
    atT = act_tokens.T
    ltT = loc_tokens.T
    c0 = context_tokens[:, 0]
    c1 = context_tokens[:, 1]

    time_z, act_z, ctxT = _gather_time_act_ctx(
        ttT, atT, c0, c1, p_time, p_act, comb)
    ltT, _ = jax.lax.optimization_barrier((ltT, time_z))
    loc_z1 = _gather_loc_h0(ltT, p_loc)
    ltT, _ = jax.lax.optimization_barrier((ltT, loc_z1))
    loc_z2 = _gather_loc_h1(ltT, p_loc)

    return (
        ctxT.T[:, :6],
        _retile(time_z),
        _retile_halves(loc_z1, loc_z2),
        _retile(act_z),
    )


# speedup vs baseline: 1.0904x; 1.0904x over previous
import functools

import jax
import jax.numpy as jnp
from jax import lax
from jax.experimental import pallas as pl
from jax.experimental.pallas import tpu as pltpu, tpu_sc as plsc

B = 16384
L = 200
LH = L // 2
BL = B * L

NC = 2
NS = 16
NW = NC * NS
LANES = 16

BPW = B // NW
CTX_PAD = 16
TV = 1000
LOC_VP = 1024000


def _proj_body(xt_ref, w_ref, b_ref, o_ref):
    y = lax.dot_general(
        xt_ref[...], w_ref[...], (((0,), (1,)), ((), ())),
        preferred_element_type=jnp.float32,
    )
    o_ref[...] = y + b_ref[...]


def _project(xt, w, b_row, blk):
    d_in, v = xt.shape
    return pl.pallas_call(
        _proj_body,
        grid=(v // blk,),
        in_specs=[
            pl.BlockSpec((d_in, blk), lambda i: (0, i)),
            pl.BlockSpec((32, d_in), lambda i: (0, 0)),
            pl.BlockSpec((1, 32), lambda i: (0, 0)),
        ],
        out_specs=pl.BlockSpec((blk, 32), lambda i: (i, 0)),
        out_shape=jax.ShapeDtypeStruct((v, 32), jnp.float32),
    )(xt, w, b_row)


def _retile_body(x_ref, o_ref):
    o_ref[...] = x_ref[...].T


def _retile(z):
    x2 = z.reshape(z.shape[0] * B * 128 // 128, 128)
    o2 = pl.pallas_call(
        _retile_body,
        grid=(L // 4, 4),
        in_specs=[pl.BlockSpec((4096, 128), lambda i, j: (i * 4 + j, 0))],
        out_specs=pl.BlockSpec((128, 4096), lambda i, j: (i, j)),
        out_shape=jax.ShapeDtypeStruct((L * 32, B), jnp.float32),
    )(x2)
    return o2.reshape(L, 32, B).transpose(2, 0, 1)


def _retile_body_al(x_ref, prev_ref, o_ref):
    del prev_ref
    o_ref[...] = x_ref[...].T


def _retile_halves(z1, z2):
    o2 = None
    for half, z in ((0, z1), (1, z2)):
        x2 = z.reshape(LH * B * 32 // 128, 128)
        grid_i = LH // 4
        out_map = functools.partial(
            lambda h, i, j: (h * (LH // 4) + i, j), half)
        if half == 0:
            o2 = pl.pallas_call(
                _retile_body,
                grid=(grid_i, 4),
                in_specs=[
                    pl.BlockSpec((4096, 128), lambda i, j: (i * 4 + j, 0))],
                out_specs=pl.BlockSpec((128, 4096), out_map),
                out_shape=jax.ShapeDtypeStruct((L * 32, B), jnp.float32),
            )(x2)
        else:
            o2 = pl.pallas_call(
                _retile_body_al,
                grid=(grid_i, 4),
                in_specs=[
                    pl.BlockSpec((4096, 128), lambda i, j: (i * 4 + j, 0)),
                    pl.BlockSpec(memory_space=pltpu.MemorySpace.HBM),
                ],
                out_specs=pl.BlockSpec((128, 4096), out_map),
                out_shape=jax.ShapeDtypeStruct((L * 32, B), jnp.float32),
                input_output_aliases={1: 0},
            )(x2, o2)
    return o2.reshape(L, 32, B).transpose(2, 0, 1)


_MESH = plsc.VectorSubcoreMesh(core_axis_name="c", subcore_axis_name="s")
_SC_PARAMS = pltpu.CompilerParams(
    use_tc_tiling_on_sc=False, needs_layout_passes=False)

_SLAB = jax.ShapeDtypeStruct((L // 4, B, 128), jnp.float32)
_SLABH = jax.ShapeDtypeStruct((LH // 4, B, 128), jnp.float32)


def _slab_dst(out_hbm, l, b0, l_off=0):
    ll = l - l_off
    return out_hbm.at[ll // 4, pl.ds(b0, BPW), pl.ds(lax.rem(ll, 4) * 32, 32)]


def _ring_streams(streams, b0, l_lo, l_hi, l_off=0):
    def fire(l, h):
        for tokT, tab, _out, idx_v, rows_v, gs, _ws in streams:
            pltpu.sync_copy(tokT.at[l, pl.ds(b0, BPW)], idx_v.at[h])
            pltpu.async_copy(tab.at[idx_v.at[h]], rows_v.at[h], gs.at[h])

    def drain_fire_out(l, h):
        for _tokT, tab, out, idx_v, rows_v, gs, ws in streams:
            pltpu.make_async_copy(
                tab.at[idx_v.at[h]], rows_v.at[h], gs.at[h]).wait()
            pltpu.async_copy(
                rows_v.at[h], _slab_dst(out, l, b0, l_off), ws.at[h])

    def wait_out(l, h):
        for _tokT, _tab, out, _idx_v, rows_v, _gs, ws in streams:
            pltpu.make_async_copy(
                rows_v.at[h], _slab_dst(out, l, b0, l_off), ws.at[h]).wait()

    for h in range(2):
        fire(l_lo + h, h)
    for h in range(2):
        drain_fire_out(l_lo + h, h)

    @pl.loop(l_lo + 2, l_hi, step=2)
    def _rows(i):
        for h in range(2):
            wait_out(i + h - 2, h)
            fire(i + h, h)
        for h in range(2):
            drain_fire_out(i + h, h)

    for h in range(2):
        wait_out(l_hi - 2 + h, h)


def _transpose_chunk16(rows, trows, c):
    giota = lax.iota(jnp.int32, 16)
    for d in range(CTX_PAD):
        dvec = jnp.full((16,), d, jnp.int32)
        for g in range(c // LANES):
            rvec = giota + (g * LANES)
            trows[d, pl.ds(g * LANES, LANES)] = plsc.load_gather(
                rows, [rvec, dvec])


@functools.partial(
    pl.kernel,
    mesh=_MESH,
    compiler_params=_SC_PARAMS,
    out_type=[
        _SLAB,
        _SLAB,
        jax.ShapeDtypeStruct((CTX_PAD, B), jnp.float32),
    ],
    scratch_types=[
        pltpu.VMEM_SHARED((TV, 32), jnp.float32),
        pltpu.VMEM_SHARED((TV, 32), jnp.float32),
        pltpu.VMEM_SHARED((304, CTX_PAD), jnp.float32),
        pltpu.VMEM((2, BPW), jnp.int32),
        pltpu.VMEM((2, BPW, 32), jnp.float32),
        pltpu.VMEM((2, BPW), jnp.int32),
        pltpu.VMEM((2, BPW, 32), jnp.float32),
        pltpu.VMEM((BPW,), jnp.int32),
        pltpu.VMEM((BPW,), jnp.int32),
        pltpu.VMEM((BPW,), jnp.int32),
        pltpu.VMEM((BPW, CTX_PAD), jnp.float32),
        pltpu.VMEM((CTX_PAD, BPW), jnp.float32),
        pltpu.SemaphoreType.DMA((2,)),
        pltpu.SemaphoreType.DMA((2,)),
        pltpu.SemaphoreType.DMA((2,)),
        pltpu.SemaphoreType.DMA((2,)),
        pltpu.SemaphoreType.DMA,
    ],
)
def _gather_time_act_ctx(ttT_hbm, atT_hbm, c0_hbm, c1_hbm,
                         p_time_hbm, p_act_hbm, p_ctx_hbm,
                         out_t_hbm, out_a_hbm, out_c_hbm,
                         ptime_sh, pact_sh, comb_sh,
                         t_idx, t_rows, a_idx, a_rows,
                         c0_v, c1_v, cidx_v, crows_v, ctr_v,
                         tg, to, ag, ao, csem):
    cid = lax.axis_index("c")
    sid = lax.axis_index("s")
    wid = sid * NC + cid
    b0 = wid * BPW

    @pl.when(sid == 0)
    def _stage():
        pltpu.sync_copy(p_time_hbm, ptime_sh)
        pltpu.sync_copy(p_act_hbm, pact_sh)
        pltpu.sync_copy(p_ctx_hbm, comb_sh)

    plsc.subcore_barrier()

    pltpu.sync_copy(c0_hbm.at[pl.ds(b0, BPW)], c0_v)
    pltpu.sync_copy(c1_hbm.at[pl.ds(b0, BPW)], c1_v)
    for j in range(BPW // LANES):
        sl = pl.ds(j * LANES, LANES)
        cidx_v[sl] = c0_v[sl] * 100 + c1_v[sl]
    pltpu.async_copy(comb_sh.at[cidx_v], crows_v, csem).wait()
    _transpose_chunk16(crows_v, ctr_v, BPW)
    pltpu.sync_copy(ctr_v, out_c_hbm.at[:, pl.ds(b0, BPW)])

    _ring_streams(
        [
            (ttT_hbm, ptime_sh, out_t_hbm, t_idx, t_rows, tg, to),
            (atT_hbm, pact_sh, out_a_hbm, a_idx, a_rows, ag, ao),
        ],
        b0, 0, L,
    )


@functools.partial(
    pl.kernel,
    mesh=_MESH,
    compiler_params=_SC_PARAMS,
    out_type=_SLAB,
    scratch_types=[
        pltpu.VMEM((2, BPW), jnp.int32),
        pltpu.VMEM((2, BPW, 32), jnp.float32),
        pltpu.SemaphoreType.DMA((2,)),
        pltpu.SemaphoreType.DMA((2,)),
    ],
)
def _gather_loc(ltT_hbm, p_loc_hbm, out_hbm, idx_v, rows_v, gs, os_):
    wid = lax.axis_index("s") * NC + lax.axis_index("c")
    b0 = wid * BPW
    _ring_streams(
        [(ltT_hbm, p_loc_hbm, out_hbm, idx_v, rows_v, gs, os_)],
        b0, 0, L,
    )


def kernel(context_tokens, time_tokens, loc_tokens, act_tokens,
           time_table, loc_table, act_table, age_table, gender_table,
           W_time, b_time, W_loc, b_loc, W_act, b_act):
    p_time = _project(time_table.T, W_time, b_time.reshape(1, 32), TV)
    p_act = _project(act_table.T, W_act, b_act.reshape(1, 32), TV)
    ltp = jnp.pad(loc_table.T, ((0, 0), (0, LOC_VP - 1000000)))
    p_loc = _project(ltp, W_loc, b_loc.reshape(1, 32), 12800)

    comb = jnp.zeros((304, CTX_PAD), jnp.float32)
    comb = comb.at[:300, :2].set(jnp.repeat(gender_table, 100, axis=0))
    comb = comb.at[:300, 2:6].set(jnp.tile(age_table, (3, 1)))

    ttT = time_tokens.T
    atT = act_tokens.T
    ltT = loc_tokens.T
    c0 = context_tokens[:, 0]
    c1 = context_tokens[:, 1]

    time_z, act_z, ctxT = _gather_time_act_ctx(
        ttT, atT, c0, c1, p_time, p_act, comb)
    ltT, _ = jax.lax.optimization_barrier((ltT, time_z))
    loc_z = _gather_loc(ltT, p_loc)

    return (
        ctxT.T[:, :6],
        _retile(time_z),
        _retile(loc_z),
        _retile(act_z),
    )


# speedup vs baseline: 1.1389x; 1.0444x over previous
import functools

import jax
import jax.numpy as jnp
from jax import lax
from jax.experimental import pallas as pl
from jax.experimental.pallas import tpu as pltpu, tpu_sc as plsc

B = 16384
L = 200
BL = B * L

NC = 2
NS = 16
NW = NC * NS
LANES = 16

BPW = B // NW
CTX_PAD = 16
TV = 1000
LOC_VP = 1024000


def _proj_body(xt_ref, w_ref, b_ref, o_ref):
    y = lax.dot_general(
        xt_ref[...], w_ref[...], (((0,), (1,)), ((), ())),
        preferred_element_type=jnp.float32,
    )
    o_ref[...] = y + b_ref[...]


def _project(xt, w, b_row, blk):
    d_in, v = xt.shape
    return pl.pallas_call(
        _proj_body,
        grid=(v // blk,),
        in_specs=[
            pl.BlockSpec((d_in, blk), lambda i: (0, i)),
            pl.BlockSpec((32, d_in), lambda i: (0, 0)),
            pl.BlockSpec((1, 32), lambda i: (0, 0)),
        ],
        out_specs=pl.BlockSpec((blk, 32), lambda i: (i, 0)),
        out_shape=jax.ShapeDtypeStruct((v, 32), jnp.float32),
    )(xt, w, b_row)


def _retile_body(x_ref, o_ref):
    o_ref[...] = x_ref[...].T


def _retile(z):
    x2 = z.reshape(z.shape[0] * B * 128 // 128, 128)
    o2 = pl.pallas_call(
        _retile_body,
        grid=(L // 4, 2),
        in_specs=[pl.BlockSpec((8192, 128), lambda i, j: (i * 2 + j, 0))],
        out_specs=pl.BlockSpec((128, 8192), lambda i, j: (i, j)),
        out_shape=jax.ShapeDtypeStruct((L * 32, B), jnp.float32),
    )(x2)
    return o2.reshape(L, 32, B).transpose(2, 0, 1)


_MESH = plsc.VectorSubcoreMesh(core_axis_name="c", subcore_axis_name="s")
_SC_PARAMS = pltpu.CompilerParams(
    use_tc_tiling_on_sc=False, needs_layout_passes=False)

_SLAB = jax.ShapeDtypeStruct((L // 4, B, 128), jnp.float32)


def _slab_dst(out_hbm, l, b0, l_off=0):
    ll = l - l_off
    return out_hbm.at[ll // 4, pl.ds(b0, BPW), pl.ds(lax.rem(ll, 4) * 32, 32)]


def _ring_streams(streams, b0, l_lo, l_hi, l_off=0):
    def fire(l, h):
        for tokT, tab, _out, idx_v, rows_v, gs, _ws in streams:
            pltpu.sync_copy(tokT.at[l, pl.ds(b0, BPW)], idx_v.at[h])
            pltpu.async_copy(tab.at[idx_v.at[h]], rows_v.at[h], gs.at[h])

    def drain_fire_out(l, h):
        for _tokT, tab, out, idx_v, rows_v, gs, ws in streams:
            pltpu.make_async_copy(
                tab.at[idx_v.at[h]], rows_v.at[h], gs.at[h]).wait()
            pltpu.async_copy(
                rows_v.at[h], _slab_dst(out, l, b0, l_off), ws.at[h])

    def wait_out(l, h):
        for _tokT, _tab, out, _idx_v, rows_v, _gs, ws in streams:
            pltpu.make_async_copy(
                rows_v.at[h], _slab_dst(out, l, b0, l_off), ws.at[h]).wait()

    for h in range(2):
        fire(l_lo + h, h)
    for h in range(2):
        drain_fire_out(l_lo + h, h)

    @pl.loop(l_lo + 2, l_hi, step=2)
    def _rows(i):
        for h in range(2):
            wait_out(i + h - 2, h)
            fire(i + h, h)
        for h in range(2):
            drain_fire_out(i + h, h)

    for h in range(2):
        wait_out(l_hi - 2 + h, h)


def _transpose_chunk16(rows, trows, c):
    giota = lax.iota(jnp.int32, 16)
    for d in range(CTX_PAD):
        dvec = jnp.full((16,), d, jnp.int32)
        for g in range(c // LANES):
            rvec = giota + (g * LANES)
            trows[d, pl.ds(g * LANES, LANES)] = plsc.load_gather(
                rows, [rvec, dvec])


@functools.partial(
    pl.kernel,
    mesh=_MESH,
    compiler_params=_SC_PARAMS,
    out_type=[
        _SLAB,
        _SLAB,
        jax.ShapeDtypeStruct((CTX_PAD, B), jnp.float32),
    ],
    scratch_types=[
        pltpu.VMEM_SHARED((TV, 32), jnp.float32),
        pltpu.VMEM_SHARED((TV, 32), jnp.float32),
        pltpu.VMEM_SHARED((304, CTX_PAD), jnp.float32),
        pltpu.VMEM((2, BPW), jnp.int32),
        pltpu.VMEM((2, BPW, 32), jnp.float32),
        pltpu.VMEM((2, BPW), jnp.int32),
        pltpu.VMEM((2, BPW, 32), jnp.float32),
        pltpu.VMEM((BPW,), jnp.int32),
        pltpu.VMEM((BPW,), jnp.int32),
        pltpu.VMEM((BPW,), jnp.int32),
        pltpu.VMEM((BPW, CTX_PAD), jnp.float32),
        pltpu.VMEM((CTX_PAD, BPW), jnp.float32),
        pltpu.SemaphoreType.DMA((2,)),
        pltpu.SemaphoreType.DMA((2,)),
        pltpu.SemaphoreType.DMA((2,)),
        pltpu.SemaphoreType.DMA((2,)),
        pltpu.SemaphoreType.DMA,
    ],
)
def _gather_time_act_ctx(ttT_hbm, atT_hbm, c0_hbm, c1_hbm,
                         p_time_hbm, p_act_hbm, p_ctx_hbm,
                         out_t_hbm, out_a_hbm, out_c_hbm,
                         ptime_sh, pact_sh, comb_sh,
                         t_idx, t_rows, a_idx, a_rows,
                         c0_v, c1_v, cidx_v, crows_v, ctr_v,
                         tg, to, ag, ao, csem):
    cid = lax.axis_index("c")
    sid = lax.axis_index("s")
    wid = sid * NC + cid
    b0 = wid * BPW

    @pl.when(sid == 0)
    def _stage():
        pltpu.sync_copy(p_time_hbm, ptime_sh)
        pltpu.sync_copy(p_act_hbm, pact_sh)
        pltpu.sync_copy(p_ctx_hbm, comb_sh)

    plsc.subcore_barrier()

    pltpu.sync_copy(c0_hbm.at[pl.ds(b0, BPW)], c0_v)
    pltpu.sync_copy(c1_hbm.at[pl.ds(b0, BPW)], c1_v)
    for j in range(BPW // LANES):
        sl = pl.ds(j * LANES, LANES)
        cidx_v[sl] = c0_v[sl] * 100 + c1_v[sl]
    pltpu.async_copy(comb_sh.at[cidx_v], crows_v, csem).wait()
    _transpose_chunk16(crows_v, ctr_v, BPW)
    pltpu.sync_copy(ctr_v, out_c_hbm.at[:, pl.ds(b0, BPW)])

    _ring_streams(
        [
            (ttT_hbm, ptime_sh, out_t_hbm, t_idx, t_rows, tg, to),
            (atT_hbm, pact_sh, out_a_hbm, a_idx, a_rows, ag, ao),
        ],
        b0, 0, L,
    )


@functools.partial(
    pl.kernel,
    mesh=_MESH,
    compiler_params=_SC_PARAMS,
    out_type=_SLAB,
    scratch_types=[
        pltpu.VMEM((2, BPW), jnp.int32),
        pltpu.VMEM((2, BPW, 32), jnp.float32),
        pltpu.SemaphoreType.DMA((2,)),
        pltpu.SemaphoreType.DMA((2,)),
    ],
)
def _gather_loc(ltT_hbm, p_loc_hbm, out_hbm, idx_v, rows_v, gs, os_):
    wid = lax.axis_index("s") * NC + lax.axis_index("c")
    b0 = wid * BPW
    _ring_streams(
        [(ltT_hbm, p_loc_hbm, out_hbm, idx_v, rows_v, gs, os_)],
        b0, 0, L,
    )


def kernel(context_tokens, time_tokens, loc_tokens, act_tokens,
           time_table, loc_table, act_table, age_table, gender_table,
           W_time, b_time, W_loc, b_loc, W_act, b_act):
    p_time = _project(time_table.T, W_time, b_time.reshape(1, 32), TV)
    p_act = _project(act_table.T, W_act, b_act.reshape(1, 32), TV)
    ltp = jnp.pad(loc_table.T, ((0, 0), (0, LOC_VP - 1000000)))
    p_loc = _project(ltp, W_loc, b_loc.reshape(1, 32), 12800)

    comb = jnp.zeros((304, CTX_PAD), jnp.float32)
    comb = comb.at[:300, :2].set(jnp.repeat(gender_table, 100, axis=0))
    comb = comb.at[:300, 2:6].set(jnp.tile(age_table, (3, 1)))

    ttT = time_tokens.T
    atT = act_tokens.T
    ltT = loc_tokens.T
    c0 = context_tokens[:, 0]
    c1 = context_tokens[:, 1]

    time_z, act_z, ctxT = _gather_time_act_ctx(
        ttT, atT, c0, c1, p_time, p_act, comb)
    ltT, _ = jax.lax.optimization_barrier((ltT, time_z))
    loc_z = _gather_loc(ltT, p_loc)

    return (
        ctxT.T[:, :6],
        _retile(time_z),
        _retile(loc_z),
        _retile(act_z),
    )
